# baseline (speedup 1.0000x reference)
import functools

import jax
import jax.numpy as jnp
from jax import lax
from jax.experimental import pallas as pl
from jax.experimental.pallas import tpu as pltpu
from jax.experimental.pallas import tpu_sc as plsc

N = 10000
NP = 10240
D = 128
E = 320000
EP = 327680
CR = EP // 128
NC, NS = 2, 16
NW = NC * NS
RPT = CR // NW
RPS = CR // NS
JUNK = 10200
ART = NP // NS


def _mesh():
    return plsc.VectorSubcoreMesh(core_axis_name="c", subcore_axis_name="s",
                                  num_cores=NC, num_subcores=NS)


def _deg_body(play1, play2, soc, counts,
              idx_v, ones_v, cnt_v, acc0, acc1, acc2, csem):
    cid = lax.axis_index("c")
    sid = lax.axis_index("s")
    ones16 = jnp.ones((16,), jnp.float32)
    zero16 = jnp.zeros((16,), jnp.float32)

    def initbuf(i, _):
        ones_v[i, :] = ones16
        cnt_v[i, :] = zero16
        return 0
    lax.fori_loop(0, 128, initbuf, 0)

    for a in (acc0, acc1, acc2):
        def zacc(k, _, a=a):
            pltpu.sync_copy(cnt_v, a.at[pl.ds(sid * ART + k * 128, 128)])
            return 0
        lax.fori_loop(0, 5, zacc, 0)
    plsc.subcore_barrier()

    def count_into(a, earr, col):
        pltpu.sync_copy(earr.at[col, pl.ds(sid * RPS, RPS)], idx_v)

        def batch(t, _):
            for b in range(8):
                pltpu.async_copy(ones_v, a.at[idx_v.at[t * 8 + b]], csem,
                                 add=True)
            for _b in range(8):
                pltpu.make_async_copy(ones_v, a.at[pl.ds(0, 128)], csem).wait()
            return 0
        lax.fori_loop(0, RPS // 8, batch, 0)

    @pl.when(cid == 0)
    def _():
        count_into(acc0, play1, 0)
        count_into(acc1, play1, 1)
        count_into(acc2, play2, 0)

    @pl.when(cid == 1)
    def _():
        count_into(acc0, play2, 1)
        count_into(acc1, soc, 0)
        count_into(acc2, soc, 1)

    plsc.subcore_barrier()

    def fin(a, r):
        sl = pl.ds(sid * ART, ART)
        pltpu.sync_copy(a.at[sl], counts.at[r, sl])

    @pl.when(cid == 0)
    def _():
        fin(acc0, 0)
        fin(acc1, 1)
        fin(acc2, 2)

    @pl.when(cid == 1)
    def _():
        fin(acc0, 3)
        fin(acc1, 4)
        fin(acc2, 5)


@functools.cache
def _deg_kernel_build():
    return pl.kernel(
        _deg_body,
        out_type=jax.ShapeDtypeStruct((6, NP, 16), jnp.float32),
        mesh=_mesh(),
        compiler_params=pltpu.CompilerParams(use_tc_tiling_on_sc=False),
        scratch_types=[
            pltpu.VMEM((RPS, 128), jnp.int32),
            pltpu.VMEM((128, 16), jnp.float32),
            pltpu.VMEM((128, 16), jnp.float32),
            pltpu.VMEM_SHARED((NP, 16), jnp.float32),
            pltpu.VMEM_SHARED((NP, 16), jnp.float32),
            pltpu.VMEM_SHARED((NP, 16), jnp.float32),
            pltpu.SemaphoreType.DMA,
        ],
    )


def _weights_body(invU1, invG1, invU2, invG2, invSO, invSI,
                  play1, play2, soc, nw1, di1, ci1, nw2, di2, ci2,
                  wA1, wB1, wA2, wB2, wS1, wS2,
                  iu, ig, a0, b0, p0, q0, r0, s0, a1, b1, p1, q1, r1, s1,
                  gsem, wsem, osem):
    cid = lax.axis_index("c")
    sid = lax.axis_index("s")
    wid = cid * NS + sid
    sl = pl.ds(wid * RPT, RPT)
    A, B = (a0, a1), (b0, b1)
    W1, W2, W3, W4 = (p0, p1), (q0, q1), (r0, r1), (s0, s1)

    def ebase(j):
        return pl.ds((wid * RPT + j) * 128, 128)

    def run_pass(invA, invB, warrs, outs, compute):
        wbufs = (W1, W2, W3, W4)[:len(warrs)]
        obufs = ((A, B) if len(outs) == 2 and len(warrs) == 1
                 else (W1, W3))[:len(outs)]

        def fire_in(j, s):
            pltpu.async_copy(invA.at[iu.at[j]], A[s], gsem)
            pltpu.async_copy(invB.at[ig.at[j]], B[s], gsem)
            for w, wb in zip(warrs, wbufs):
                pltpu.async_copy(w.at[ebase(j)], wb[s], wsem)

        def drain_in(s):
            pltpu.make_async_copy(invA.at[pl.ds(0, 128)], A[s], gsem).wait()
            pltpu.make_async_copy(invA.at[pl.ds(0, 128)], B[s], gsem).wait()
            for w, wb in zip(warrs, wbufs):
                pltpu.make_async_copy(w.at[pl.ds(0, 128)], wb[s], wsem).wait()

        def fire_out(j, s):
            for o, ob in zip(outs, obufs):
                pltpu.async_copy(ob[s], o.at[ebase(j)], osem)

        def drain_out():
            for o, ob in zip(outs, obufs):
                pltpu.make_async_copy(ob[0], o.at[pl.ds(0, 128)], osem).wait()

        fire_in(0, 0)
        fire_in(1, 1)
        drain_in(0)
        compute(0)
        fire_out(0, 0)

        def pair(t, _):
            for s5 in range(2):
                j = 1 + 2 * t + s5
                s = (1 + s5) % 2
                drain_out()
                fire_in(j + 1, 1 - s)
                drain_in(s)
                compute(s)
                fire_out(j, s)
            return 0
        lax.fori_loop(0, (RPT - 2) // 2, pair, 0)

        drain_out()
        drain_in(1)
        compute(1)
        fire_out(RPT - 1, 1)
        drain_out()

    def play_compute(s):
        def row(e, _):
            t = A[s][e, :] * B[s][e, :]
            B[s][e, :] = t
            A[s][e, :] = t * (1.0 + 0.1 * W1[s][e, :])
            return 0
        lax.fori_loop(0, 128, row, 0, unroll=2)

    def soc_compute(s):
        def row(e, _):
            m = A[s][e, :] * B[s][e, :]
            W1[s][e, :] = 0.2 * (W1[s][e, :] + W2[s][e, :]) * m
            W3[s][e, :] = 0.2 * (W3[s][e, :] + W4[s][e, :]) * m
            return 0
        lax.fori_loop(0, 128, row, 0, unroll=2)

    pltpu.sync_copy(play1.at[0, sl], iu)
    pltpu.sync_copy(play1.at[1, sl], ig)
    run_pass(invU1, invG1, (nw1,), (wA1, wB1), play_compute)
    pltpu.sync_copy(play2.at[0, sl], iu)
    pltpu.sync_copy(play2.at[1, sl], ig)
    run_pass(invU2, invG2, (nw2,), (wA2, wB2), play_compute)
    pltpu.sync_copy(soc.at[0, sl], iu)
    pltpu.sync_copy(soc.at[1, sl], ig)
    run_pass(invSO, invSI, (di1, ci1, di2, ci2), (wS1, wS2), soc_compute)


@functools.cache
def _weights_kernel_build():
    return pl.kernel(
        _weights_body,
        out_type=tuple(jax.ShapeDtypeStruct((EP, 16), jnp.float32)
                       for _ in range(6)),
        mesh=_mesh(),
        compiler_params=pltpu.CompilerParams(use_tc_tiling_on_sc=False),
        scratch_types=(
            [pltpu.VMEM((RPT, 128), jnp.int32)] * 2
            + [pltpu.VMEM((128, 16), jnp.float32)] * 12
            + [pltpu.SemaphoreType.DMA] * 3
        ),
    )


DH = 64
_RING = 5


def _spmm_phase(tab, earr, scol, dcol, warr, wid, sidx, didx, rows, wv,
                acc, gsem, wsem, ssem):
    esl = pl.ds(wid * RPT, RPT)
    pltpu.sync_copy(earr.at[scol, esl], sidx)
    pltpu.sync_copy(earr.at[dcol, esl], didx)

    def fire(j, s):
        pltpu.async_copy(tab.at[sidx.at[j]], rows[s], gsem)
        pltpu.async_copy(warr.at[pl.ds((wid * RPT + j) * 128, 128)], wv[s], wsem)

    def drain_g(s):
        pltpu.make_async_copy(tab.at[pl.ds(0, 128)], rows[s], gsem).wait()
        pltpu.make_async_copy(warr.at[pl.ds(0, 128)], wv[s], wsem).wait()

    def mul(s):
        def mb(e, _):
            w = wv[s][e, :]
            for q in range(DH // 16):
                sl16 = pl.ds(q * 16, 16)
                rows[s][e, sl16] = rows[s][e, sl16] * w
            return 0
        lax.fori_loop(0, 128, mb, 0, unroll=4)

    def fire_sc(j, s):
        pltpu.async_copy(rows[s], acc.at[didx.at[j]], ssem, add=True)

    def drain_sc():
        pltpu.make_async_copy(rows[0], acc.at[pl.ds(0, 128)], ssem).wait()

    for s in range(3):
        fire(s, s)
    for j in (0, 1):
        drain_g(j)
        mul(j)
        fire_sc(j, j)
        fire(j + 3, j + 3)

    def quint(t, _):
        j0 = 2 + t * 5
        for s5 in range(5):
            j = j0 + s5
            s = (2 + s5) % 5
            drain_g(s)
            mul(s)
            fire_sc(j, s)
            drain_sc()
            fire(j + 3, s5)
        return 0
    lax.fori_loop(0, 15, quint, 0)

    for j, s in ((77, 2), (78, 3), (79, 4)):
        drain_g(s)
        mul(s)
        fire_sc(j, s)
    for _ in range(5):
        drain_sc()


def _spmm_half(phases, out, wid, sid, sidx, didx, rows, wv, acc,
               gsem, wsem, ssem):
    zero16 = jnp.zeros((16,), jnp.float32)
    r0 = rows[0]

    def zr(e, _):
        for q in range(DH // 16):
            r0[e, pl.ds(q * 16, 16)] = zero16
        return 0
    lax.fori_loop(0, 128, zr, 0)

    def za(k, _):
        pltpu.sync_copy(r0, acc.at[pl.ds(sid * ART + k * 128, 128)])
        return 0
    lax.fori_loop(0, ART // 128, za, 0)
    plsc.subcore_barrier()

    for (tab, earr, scol, dcol, warr) in phases:
        _spmm_phase(tab, earr, scol, dcol, warr, wid, sidx, didx, rows, wv,
                    acc, gsem, wsem, ssem)

    plsc.subcore_barrier()
    cid = lax.axis_index("c")
    pltpu.sync_copy(acc.at[pl.ds(sid * ART, ART)],
                    out.at[cid, pl.ds(sid * ART, ART)])
    plsc.subcore_barrier()


def _spmm_body(phasesL, phasesR, outL, outR, sidx, didx,
               r0, r1, r2, r3, r4, w0, w1, w2, w3, w4, acc, gsem, wsem, ssem):
    cid = lax.axis_index("c")
    sid = lax.axis_index("s")
    wid = cid * NS + sid
    rows = (r0, r1, r2, r3, r4)
    wv = (w0, w1, w2, w3, w4)
    _spmm_half(phasesL, outL, wid, sid, sidx, didx, rows, wv, acc,
               gsem, wsem, ssem)
    _spmm_half(phasesR, outR, wid, sid, sidx, didx, rows, wv, acc,
               gsem, wsem, ssem)


def _spmm2_body(hgL, hgR, huL, huR, e1, e2, w1, w2, outL, outR, *sc):
    _spmm_body([(hgL, e1, 1, 0, w1), (huL, e2, 0, 1, w2)],
               [(hgR, e1, 1, 0, w1), (huR, e2, 0, 1, w2)],
               outL, outR, *sc)


def _spmm1_body(huL, huR, e1, w1, outL, outR, *sc):
    _spmm_body([(huL, e1, 0, 1, w1)], [(huR, e1, 0, 1, w1)],
               outL, outR, *sc)


@functools.cache
def _spmm_build(nphases):
    body = _spmm2_body if nphases == 2 else _spmm1_body
    return pl.kernel(
        body,
        out_type=(jax.ShapeDtypeStruct((NC, NP, DH), jnp.float32),
                  jax.ShapeDtypeStruct((NC, NP, DH), jnp.float32)),
        mesh=_mesh(),
        compiler_params=pltpu.CompilerParams(use_tc_tiling_on_sc=False),
        scratch_types=(
            [pltpu.VMEM((RPT, 128), jnp.int32)] * 2
            + [pltpu.VMEM((128, DH), jnp.float32)] * 5
            + [pltpu.VMEM((128, 16), jnp.float32)] * 5
            + [pltpu.VMEM_SHARED((NP, DH), jnp.float32)]
            + [pltpu.SemaphoreType.DMA] * 3
        ),
    )


def _invsq_body(c_ref, o_ref):
    o_ref[...] = jax.lax.rsqrt(jnp.maximum(c_ref[...], 1.0))


_invsq_tc = pl.pallas_call(
    _invsq_body,
    out_shape=jax.ShapeDtypeStruct((6 * NP * 16 // 128, 128), jnp.float32),
)


_BM = 1024


def _merge_body(ul_ref, ur_ref, gl_ref, gr_ref, hul, hur, hgl, hgr):
    hul[...] = ul_ref[0] + ul_ref[1]
    hur[...] = ur_ref[0] + ur_ref[1]
    hgl[...] = gl_ref[0] + gl_ref[1]
    hgr[...] = gr_ref[0] + gr_ref[1]


_merge = pl.pallas_call(
    _merge_body,
    grid=(NP // _BM,),
    in_specs=[pl.BlockSpec((NC, _BM, DH), lambda i: (0, i, 0))] * 4,
    out_specs=[pl.BlockSpec((_BM, DH), lambda i: (i, 0))] * 4,
    out_shape=[jax.ShapeDtypeStruct((NP, DH), jnp.float32)] * 4,
)


def _pad_edges(e):
    e = e.astype(jnp.int32)
    pad = jnp.full((2, EP - E), JUNK, jnp.int32)
    return jnp.concatenate([e, pad], axis=1).reshape(2, CR, 128)


def _rep_w(w):
    w = jnp.concatenate([w.astype(jnp.float32), jnp.zeros((EP - E,), jnp.float32)])
    return jnp.broadcast_to(w[:, None], (EP, 16))


def _pad_tab(t):
    return jnp.concatenate(
        [t.astype(jnp.float32), jnp.zeros((NP - N, D), jnp.float32)], axis=0)


def kernel(user_embedding, item_embedding, weight_noise_edge, weight_friend_of_DI,
           weight_friend_of_CI, dn_weight_noise_edge, dn_weight_friend_of_DI,
           dn_weight_friend_of_CI, edge_index_play, edge_index_social,
           edge_index_play_dn):
    p1 = _pad_edges(edge_index_play)
    p2 = _pad_edges(edge_index_play_dn)
    so = _pad_edges(edge_index_social)
    nw1 = _rep_w(weight_noise_edge)
    di1 = _rep_w(weight_friend_of_DI)
    ci1 = _rep_w(weight_friend_of_CI)
    nw2 = _rep_w(dn_weight_noise_edge)
    di2 = _rep_w(dn_weight_friend_of_DI)
    ci2 = _rep_w(dn_weight_friend_of_CI)

    counts = _deg_kernel_build()(p1, p2, so)
    invsq = _invsq_tc(counts.reshape(6 * NP * 16 // 128, 128)).reshape(6, NP, 16)
    wA1, wB1, wA2, wB2, wS1, wS2 = _weights_kernel_build()(
        invsq[0], invsq[1], invsq[2], invsq[3], invsq[4], invsq[5],
        p1, p2, so, nw1, di1, ci1, nw2, di2, ci2)
    _spmm2 = _spmm_build(2)
    _spmm1 = _spmm_build(1)

    hu0 = _pad_tab(user_embedding)
    hg0 = _pad_tab(item_embedding)
    outs = []
    for (p, wA, wB, wS) in ((p1, wA1, wB1, wS1), (p2, wA2, wB2, wS2)):
        huL, huR = hu0[:, :DH], hu0[:, DH:]
        hgL, hgR = hg0[:, :DH], hg0[:, DH:]
        for _ in range(3):
            upL, upR = _spmm2(hgL, hgR, huL, huR, p, so, wA, wS)
            gpL, gpR = _spmm1(huL, huR, p, wB)
            huL, huR, hgL, hgR = _merge(upL, upR, gpL, gpR)
        outs += [jnp.concatenate([huL[:N], huR[:N]], axis=1),
                 jnp.concatenate([hgL[:N], hgR[:N]], axis=1)]
    return jnp.stack(outs)


# speedup vs baseline: 1.7856x; 1.7856x over previous
import functools

import jax
import jax.numpy as jnp
from jax import lax
from jax.experimental import pallas as pl
from jax.experimental.pallas import tpu as pltpu
from jax.experimental.pallas import tpu_sc as plsc

N = 10000
NP = 10240
D = 128
E = 320000
EP = 327680
CR = EP // 128
NC, NS = 2, 16
NW = NC * NS
RPT = CR // NW
RPS = CR // NS
JUNK = 10200
ART = NP // NS


def _mesh():
    return plsc.VectorSubcoreMesh(core_axis_name="c", subcore_axis_name="s",
                                  num_cores=NC, num_subcores=NS)


def _deg_body(play1, play2, soc, counts,
              idx_v, ones_v, cnt_v, acc0, acc1, acc2, csem):
    cid = lax.axis_index("c")
    sid = lax.axis_index("s")
    ones16 = jnp.ones((16,), jnp.float32)
    zero16 = jnp.zeros((16,), jnp.float32)

    def initbuf(i, _):
        ones_v[i, :] = ones16
        cnt_v[i, :] = zero16
        return 0
    lax.fori_loop(0, 128, initbuf, 0)

    for a in (acc0, acc1, acc2):
        def zacc(k, _, a=a):
            pltpu.sync_copy(cnt_v, a.at[pl.ds(sid * ART + k * 128, 128)])
            return 0
        lax.fori_loop(0, 5, zacc, 0)
    plsc.subcore_barrier()

    def count_into(a, earr, col):
        pltpu.sync_copy(earr.at[col, pl.ds(sid * RPS, RPS)], idx_v)

        def batch(t, _):
            for b in range(8):
                pltpu.async_copy(ones_v, a.at[idx_v.at[t * 8 + b]], csem,
                                 add=True)
            for _b in range(8):
                pltpu.make_async_copy(ones_v, a.at[pl.ds(0, 128)], csem).wait()
            return 0
        lax.fori_loop(0, RPS // 8, batch, 0)

    @pl.when(cid == 0)
    def _():
        count_into(acc0, play1, 0)
        count_into(acc1, play1, 1)
        count_into(acc2, play2, 0)

    @pl.when(cid == 1)
    def _():
        count_into(acc0, play2, 1)
        count_into(acc1, soc, 0)
        count_into(acc2, soc, 1)

    plsc.subcore_barrier()

    def fin(a, r):
        sl = pl.ds(sid * ART, ART)
        pltpu.sync_copy(a.at[sl], counts.at[r, sl])

    @pl.when(cid == 0)
    def _():
        fin(acc0, 0)
        fin(acc1, 1)
        fin(acc2, 2)

    @pl.when(cid == 1)
    def _():
        fin(acc0, 3)
        fin(acc1, 4)
        fin(acc2, 5)


@functools.cache
def _deg_kernel_build():
    return pl.kernel(
        _deg_body,
        out_type=jax.ShapeDtypeStruct((6, NP, 16), jnp.float32),
        mesh=_mesh(),
        compiler_params=pltpu.CompilerParams(use_tc_tiling_on_sc=False),
        scratch_types=[
            pltpu.VMEM((RPS, 128), jnp.int32),
            pltpu.VMEM((128, 16), jnp.float32),
            pltpu.VMEM((128, 16), jnp.float32),
            pltpu.VMEM_SHARED((NP, 16), jnp.float32),
            pltpu.VMEM_SHARED((NP, 16), jnp.float32),
            pltpu.VMEM_SHARED((NP, 16), jnp.float32),
            pltpu.SemaphoreType.DMA,
        ],
    )


def _weights_body(invU1, invG1, invU2, invG2, invSO, invSI,
                  play1, play2, soc, nw1, di1, ci1, nw2, di2, ci2,
                  wA1, wB1, wA2, wB2, wS1, wS2,
                  iu, ig, a0, b0, p0, q0, r0, s0, a1, b1, p1, q1, r1, s1,
                  gsem, wsem, osem):
    cid = lax.axis_index("c")
    sid = lax.axis_index("s")
    wid = cid * NS + sid
    sl = pl.ds(wid * RPT, RPT)
    A, B = (a0, a1), (b0, b1)
    W1, W2, W3, W4 = (p0, p1), (q0, q1), (r0, r1), (s0, s1)

    def ebase(j):
        return pl.ds((wid * RPT + j) * 128, 128)

    def run_pass(invA, invB, warrs, outs, compute):
        wbufs = (W1, W2, W3, W4)[:len(warrs)]
        obufs = ((A, B) if len(outs) == 2 and len(warrs) == 1
                 else (W1, W3))[:len(outs)]

        def fire_in(j, s):
            pltpu.async_copy(invA.at[iu.at[j]], A[s], gsem)
            pltpu.async_copy(invB.at[ig.at[j]], B[s], gsem)
            for w, wb in zip(warrs, wbufs):
                pltpu.async_copy(w.at[ebase(j)], wb[s], wsem)

        def drain_in(s):
            pltpu.make_async_copy(invA.at[pl.ds(0, 128)], A[s], gsem).wait()
            pltpu.make_async_copy(invA.at[pl.ds(0, 128)], B[s], gsem).wait()
            for w, wb in zip(warrs, wbufs):
                pltpu.make_async_copy(w.at[pl.ds(0, 128)], wb[s], wsem).wait()

        def fire_out(j, s):
            for o, ob in zip(outs, obufs):
                pltpu.async_copy(ob[s], o.at[ebase(j)], osem)

        def drain_out():
            for o, ob in zip(outs, obufs):
                pltpu.make_async_copy(ob[0], o.at[pl.ds(0, 128)], osem).wait()

        fire_in(0, 0)
        fire_in(1, 1)
        drain_in(0)
        compute(0)
        fire_out(0, 0)

        def pair(t, _):
            for s5 in range(2):
                j = 1 + 2 * t + s5
                s = (1 + s5) % 2
                drain_out()
                fire_in(j + 1, 1 - s)
                drain_in(s)
                compute(s)
                fire_out(j, s)
            return 0
        lax.fori_loop(0, (RPT - 2) // 2, pair, 0)

        drain_out()
        drain_in(1)
        compute(1)
        fire_out(RPT - 1, 1)
        drain_out()

    def play_compute(s):
        def row(e, _):
            t = A[s][e, :] * B[s][e, :]
            B[s][e, :] = t
            A[s][e, :] = t * (1.0 + 0.1 * W1[s][e, :])
            return 0
        lax.fori_loop(0, 128, row, 0, unroll=2)

    def soc_compute(s):
        def row(e, _):
            m = A[s][e, :] * B[s][e, :]
            W1[s][e, :] = 0.2 * (W1[s][e, :] + W2[s][e, :]) * m
            W3[s][e, :] = 0.2 * (W3[s][e, :] + W4[s][e, :]) * m
            return 0
        lax.fori_loop(0, 128, row, 0, unroll=2)

    pltpu.sync_copy(play1.at[0, sl], iu)
    pltpu.sync_copy(play1.at[1, sl], ig)
    run_pass(invU1, invG1, (nw1,), (wA1, wB1), play_compute)
    pltpu.sync_copy(play2.at[0, sl], iu)
    pltpu.sync_copy(play2.at[1, sl], ig)
    run_pass(invU2, invG2, (nw2,), (wA2, wB2), play_compute)
    pltpu.sync_copy(soc.at[0, sl], iu)
    pltpu.sync_copy(soc.at[1, sl], ig)
    run_pass(invSO, invSI, (di1, ci1, di2, ci2), (wS1, wS2), soc_compute)


@functools.cache
def _weights_kernel_build():
    return pl.kernel(
        _weights_body,
        out_type=tuple(jax.ShapeDtypeStruct((EP, 16), jnp.float32)
                       for _ in range(6)),
        mesh=_mesh(),
        compiler_params=pltpu.CompilerParams(use_tc_tiling_on_sc=False),
        scratch_types=(
            [pltpu.VMEM((RPT, 128), jnp.int32)] * 2
            + [pltpu.VMEM((128, 16), jnp.float32)] * 12
            + [pltpu.SemaphoreType.DMA] * 3
        ),
    )


DH = 64
CK = 64
CPT = 10240 // CK
_RING = 4


def _spmm_phase(tabS, earr, scol, dcol, warr, wid, sidx, didx, rows, wv,
                acc, gsem, wsem, ssem):
    esl = pl.ds(wid * CPT, CPT)
    pltpu.sync_copy(earr.at[scol, esl], sidx)
    pltpu.sync_copy(earr.at[dcol, esl], didx)

    def fire_g(j, s):
        pltpu.async_copy(tabS.at[sidx.at[j]], rows[s], gsem)

    def fire_w(j, sw):
        pltpu.async_copy(warr.at[pl.ds((wid * CPT + j) * CK, CK)],
                         wv[sw], wsem)

    def drain_g(s):
        pltpu.make_async_copy(tabS.at[pl.ds(0, CK)], rows[s], gsem).wait()

    def drain_w(s):
        pltpu.make_async_copy(warr.at[pl.ds(0, CK)], wv[s], wsem).wait()

    def mul(s, sw):
        def mb(e, _):
            w = wv[sw][e, :]
            for q in range(DH // 16):
                sl16 = pl.ds(q * 16, 16)
                rows[s][e, sl16] = rows[s][e, sl16] * w
            return 0
        lax.fori_loop(0, CK, mb, 0, unroll=4)

    def fire_sc(j, s):
        pltpu.async_copy(rows[s], acc.at[didx.at[j]], ssem, add=True)

    def drain_sc():
        pltpu.make_async_copy(rows[0], acc.at[pl.ds(0, CK)], ssem).wait()

    def step(j, s, sw):
        drain_sc()
        drain_g(s)
        drain_w(sw)
        fire_g(j + 2, (s + 2) % 4)
        fire_w(j + 1, 1 - sw)
        mul(s, sw)
        fire_sc(j, s)

    fire_g(0, 0)
    fire_g(1, 1)
    fire_w(0, 0)
    drain_g(0)
    drain_w(0)
    fire_g(2, 2)
    fire_w(1, 1)
    mul(0, 0)
    fire_sc(0, 0)
    drain_g(1)
    drain_w(1)
    fire_g(3, 3)
    fire_w(2, 0)
    mul(1, 1)
    fire_sc(1, 1)

    def quad(t, _):
        j0 = 2 + t * 4
        for s4 in range(4):
            step(j0 + s4, (2 + s4) % 4, s4 % 2)
        return 0
    lax.fori_loop(0, (CPT - 4) // 4, quad, 0)

    drain_sc()
    drain_g(158 % 4)
    drain_w(0)
    fire_w(159, 1)
    mul(158 % 4, 0)
    fire_sc(158, 158 % 4)
    drain_sc()
    drain_g(159 % 4)
    drain_w(1)
    mul(159 % 4, 1)
    fire_sc(159, 159 % 4)
    drain_sc()
    drain_sc()


def _stage_tab(tab, tabS, sid):
    sl = pl.ds(sid * ART, ART)
    pltpu.sync_copy(tab.at[sl], tabS.at[sl])


def _spmm_half(phases, out, wid, sid, sidx, didx, rows, wv, acc, tabS,
               gsem, wsem, ssem):
    zero16 = jnp.zeros((16,), jnp.float32)
    r0 = rows[0]

    def zr(e, _):
        for q in range(DH // 16):
            r0[e, pl.ds(q * 16, 16)] = zero16
        return 0
    lax.fori_loop(0, CK, zr, 0)

    def za(k, _):
        pltpu.sync_copy(r0, acc.at[pl.ds(sid * ART + k * CK, CK)])
        return 0
    lax.fori_loop(0, ART // CK, za, 0)

    for (tab, earr, scol, dcol, warr) in phases:
        _stage_tab(tab, tabS, sid)
        plsc.subcore_barrier()
        _spmm_phase(tabS, earr, scol, dcol, warr, wid, sidx, didx, rows, wv,
                    acc, gsem, wsem, ssem)
        plsc.subcore_barrier()

    cid = lax.axis_index("c")
    pltpu.sync_copy(acc.at[pl.ds(sid * ART, ART)],
                    out.at[cid, pl.ds(sid * ART, ART)])
    plsc.subcore_barrier()


def _spmm_body(phasesL, phasesR, outL, outR, sidx, didx,
               r0, r1, r2, r3, w0, w1, acc, tabS, gsem, wsem, ssem):
    cid = lax.axis_index("c")
    sid = lax.axis_index("s")
    wid = cid * NS + sid
    rows = (r0, r1, r2, r3)
    wv = (w0, w1)
    _spmm_half(phasesL, outL, wid, sid, sidx, didx, rows, wv, acc, tabS,
               gsem, wsem, ssem)
    _spmm_half(phasesR, outR, wid, sid, sidx, didx, rows, wv, acc, tabS,
               gsem, wsem, ssem)


def _spmm2_body(hgL, hgR, huL, huR, e1, e2, w1, w2, outL, outR, *sc):
    _spmm_body([(hgL, e1, 1, 0, w1), (huL, e2, 0, 1, w2)],
               [(hgR, e1, 1, 0, w1), (huR, e2, 0, 1, w2)],
               outL, outR, *sc)


def _spmm1_body(huL, huR, e1, w1, outL, outR, *sc):
    _spmm_body([(huL, e1, 0, 1, w1)], [(huR, e1, 0, 1, w1)],
               outL, outR, *sc)


@functools.cache
def _spmm_build(nphases):
    body = _spmm2_body if nphases == 2 else _spmm1_body
    return pl.kernel(
        body,
        out_type=(jax.ShapeDtypeStruct((NC, NP, DH), jnp.float32),
                  jax.ShapeDtypeStruct((NC, NP, DH), jnp.float32)),
        mesh=_mesh(),
        compiler_params=pltpu.CompilerParams(use_tc_tiling_on_sc=False),
        scratch_types=(
            [pltpu.VMEM((CPT, CK), jnp.int32)] * 2
            + [pltpu.VMEM((CK, DH), jnp.float32)] * 4
            + [pltpu.VMEM((CK, 16), jnp.float32)] * 2
            + [pltpu.VMEM_SHARED((NP, DH), jnp.float32)]
            + [pltpu.VMEM_SHARED((NP, DH), jnp.float32)]
            + [pltpu.SemaphoreType.DMA] * 3
        ),
    )


def _invsq_body(c_ref, o_ref):
    o_ref[...] = jax.lax.rsqrt(jnp.maximum(c_ref[...], 1.0))


_invsq_tc = pl.pallas_call(
    _invsq_body,
    out_shape=jax.ShapeDtypeStruct((6 * NP * 16 // 128, 128), jnp.float32),
)


_BM = 1024


def _merge_body(ul_ref, ur_ref, gl_ref, gr_ref, hul, hur, hgl, hgr):
    hul[...] = ul_ref[0] + ul_ref[1]
    hur[...] = ur_ref[0] + ur_ref[1]
    hgl[...] = gl_ref[0] + gl_ref[1]
    hgr[...] = gr_ref[0] + gr_ref[1]


_merge = pl.pallas_call(
    _merge_body,
    grid=(NP // _BM,),
    in_specs=[pl.BlockSpec((NC, _BM, DH), lambda i: (0, i, 0))] * 4,
    out_specs=[pl.BlockSpec((_BM, DH), lambda i: (i, 0))] * 4,
    out_shape=[jax.ShapeDtypeStruct((NP, DH), jnp.float32)] * 4,
)


def _pad_edges(e):
    e = e.astype(jnp.int32)
    pad = jnp.full((2, EP - E), JUNK, jnp.int32)
    flat = jnp.concatenate([e, pad], axis=1)
    return flat.reshape(2, CR, 128), flat.reshape(2, EP // CK, CK)


def _rep_w(w):
    w = jnp.concatenate([w.astype(jnp.float32), jnp.zeros((EP - E,), jnp.float32)])
    return jnp.broadcast_to(w[:, None], (EP, 16))


def _pad_tab(t):
    return jnp.concatenate(
        [t.astype(jnp.float32), jnp.zeros((NP - N, D), jnp.float32)], axis=0)


def kernel(user_embedding, item_embedding, weight_noise_edge, weight_friend_of_DI,
           weight_friend_of_CI, dn_weight_noise_edge, dn_weight_friend_of_DI,
           dn_weight_friend_of_CI, edge_index_play, edge_index_social,
           edge_index_play_dn):
    p1, p1s = _pad_edges(edge_index_play)
    p2, p2s = _pad_edges(edge_index_play_dn)
    so, sos = _pad_edges(edge_index_social)
    nw1 = _rep_w(weight_noise_edge)
    di1 = _rep_w(weight_friend_of_DI)
    ci1 = _rep_w(weight_friend_of_CI)
    nw2 = _rep_w(dn_weight_noise_edge)
    di2 = _rep_w(dn_weight_friend_of_DI)
    ci2 = _rep_w(dn_weight_friend_of_CI)

    counts = _deg_kernel_build()(p1, p2, so)
    invsq = _invsq_tc(counts.reshape(6 * NP * 16 // 128, 128)).reshape(6, NP, 16)
    wA1, wB1, wA2, wB2, wS1, wS2 = _weights_kernel_build()(
        invsq[0], invsq[1], invsq[2], invsq[3], invsq[4], invsq[5],
        p1, p2, so, nw1, di1, ci1, nw2, di2, ci2)
    _spmm2 = _spmm_build(2)
    _spmm1 = _spmm_build(1)

    hu0 = _pad_tab(user_embedding)
    hg0 = _pad_tab(item_embedding)
    outs = []
    for (ps, wA, wB, wS) in ((p1s, wA1, wB1, wS1), (p2s, wA2, wB2, wS2)):
        huL, huR = hu0[:, :DH], hu0[:, DH:]
        hgL, hgR = hg0[:, :DH], hg0[:, DH:]
        for _ in range(3):
            upL, upR = _spmm2(hgL, hgR, huL, huR, ps, sos, wA, wS)
            gpL, gpR = _spmm1(huL, huR, ps, wB)
            huL, huR, hgL, hgR = _merge(upL, upR, gpL, gpR)
        outs += [jnp.concatenate([huL[:N], huR[:N]], axis=1),
                 jnp.concatenate([hgL[:N], hgR[:N]], axis=1)]
    return jnp.stack(outs)


# speedup vs baseline: 1.7876x; 1.0011x over previous
import functools

import jax
import jax.numpy as jnp
from jax import lax
from jax.experimental import pallas as pl
from jax.experimental.pallas import tpu as pltpu
from jax.experimental.pallas import tpu_sc as plsc

N = 10000
NP = 10240
D = 128
E = 320000
EP = 327680
CR = EP // 128
NC, NS = 2, 16
NW = NC * NS
RPT = CR // NW
RPS = CR // NS
JUNK = 10200
ART = NP // NS


def _mesh():
    return plsc.VectorSubcoreMesh(core_axis_name="c", subcore_axis_name="s",
                                  num_cores=NC, num_subcores=NS)


def _deg_body(play1, play2, soc, counts,
              idx_v, ones_v, cnt_v, acc0, acc1, acc2, csem):
    cid = lax.axis_index("c")
    sid = lax.axis_index("s")
    ones16 = jnp.ones((16,), jnp.float32)
    zero16 = jnp.zeros((16,), jnp.float32)

    def initbuf(i, _):
        ones_v[i, :] = ones16
        cnt_v[i, :] = zero16
        return 0
    lax.fori_loop(0, 128, initbuf, 0)

    for a in (acc0, acc1, acc2):
        def zacc(k, _, a=a):
            pltpu.sync_copy(cnt_v, a.at[pl.ds(sid * ART + k * 128, 128)])
            return 0
        lax.fori_loop(0, 5, zacc, 0)
    plsc.subcore_barrier()

    def count_into(a, earr, col):
        pltpu.sync_copy(earr.at[col, pl.ds(sid * RPS, RPS)], idx_v)

        def batch(t, _):
            for b in range(16):
                pltpu.async_copy(ones_v, a.at[idx_v.at[t * 16 + b]], csem,
                                 add=True)
            for _b in range(16):
                pltpu.make_async_copy(ones_v, a.at[pl.ds(0, 128)], csem).wait()
            return 0
        lax.fori_loop(0, RPS // 16, batch, 0)

    @pl.when(cid == 0)
    def _():
        count_into(acc0, play1, 0)
        count_into(acc1, play1, 1)
        count_into(acc2, play2, 0)

    @pl.when(cid == 1)
    def _():
        count_into(acc0, play2, 1)
        count_into(acc1, soc, 0)
        count_into(acc2, soc, 1)

    plsc.subcore_barrier()

    def fin(a, r):
        sl = pl.ds(sid * ART, ART)
        pltpu.sync_copy(a.at[sl], counts.at[r, sl])

    @pl.when(cid == 0)
    def _():
        fin(acc0, 0)
        fin(acc1, 1)
        fin(acc2, 2)

    @pl.when(cid == 1)
    def _():
        fin(acc0, 3)
        fin(acc1, 4)
        fin(acc2, 5)


@functools.cache
def _deg_kernel_build():
    return pl.kernel(
        _deg_body,
        out_type=jax.ShapeDtypeStruct((6, NP, 16), jnp.float32),
        mesh=_mesh(),
        compiler_params=pltpu.CompilerParams(use_tc_tiling_on_sc=False),
        scratch_types=[
            pltpu.VMEM((RPS, 128), jnp.int32),
            pltpu.VMEM((128, 16), jnp.float32),
            pltpu.VMEM((128, 16), jnp.float32),
            pltpu.VMEM_SHARED((NP, 16), jnp.float32),
            pltpu.VMEM_SHARED((NP, 16), jnp.float32),
            pltpu.VMEM_SHARED((NP, 16), jnp.float32),
            pltpu.SemaphoreType.DMA,
        ],
    )


def _weights_body(invU1, invG1, invU2, invG2, invSO, invSI,
                  play1, play2, soc, nw1, di1, ci1, nw2, di2, ci2,
                  wA1, wB1, wA2, wB2, wS1, wS2,
                  iu, ig, a0, b0, p0, q0, r0, s0, a1, b1, p1, q1, r1, s1,
                  shA, shB, gsem, wsem, osem):
    cid = lax.axis_index("c")
    sid = lax.axis_index("s")
    wid = cid * NS + sid
    sl = pl.ds(wid * RPT, RPT)
    A, B = (a0, a1), (b0, b1)
    W1, W2, W3, W4 = (p0, p1), (q0, q1), (r0, r1), (s0, s1)

    def ebase(j):
        return pl.ds((wid * RPT + j) * 128, 128)

    def run_pass(invA, invB, warrs, outs, compute):
        sl2 = pl.ds(sid * ART, ART)
        pltpu.sync_copy(invA.at[sl2], shA.at[sl2])
        pltpu.sync_copy(invB.at[sl2], shB.at[sl2])
        plsc.subcore_barrier()
        wbufs = (W1, W2, W3, W4)[:len(warrs)]
        obufs = ((A, B) if len(outs) == 2 and len(warrs) == 1
                 else (W1, W3))[:len(outs)]

        def fire_in(j, s):
            pltpu.async_copy(shA.at[iu.at[j]], A[s], gsem)
            pltpu.async_copy(shB.at[ig.at[j]], B[s], gsem)
            for w, wb in zip(warrs, wbufs):
                pltpu.async_copy(w.at[ebase(j)], wb[s], wsem)

        def drain_in(s):
            pltpu.make_async_copy(shA.at[pl.ds(0, 128)], A[s], gsem).wait()
            pltpu.make_async_copy(shA.at[pl.ds(0, 128)], B[s], gsem).wait()
            for w, wb in zip(warrs, wbufs):
                pltpu.make_async_copy(w.at[pl.ds(0, 128)], wb[s], wsem).wait()

        def fire_out(j, s):
            for o, ob in zip(outs, obufs):
                pltpu.async_copy(ob[s], o.at[ebase(j)], osem)

        def drain_out():
            for o, ob in zip(outs, obufs):
                pltpu.make_async_copy(ob[0], o.at[pl.ds(0, 128)], osem).wait()

        fire_in(0, 0)
        fire_in(1, 1)
        drain_in(0)
        compute(0)
        fire_out(0, 0)

        def pair(t, _):
            for s5 in range(2):
                j = 1 + 2 * t + s5
                s = (1 + s5) % 2
                drain_out()
                fire_in(j + 1, 1 - s)
                drain_in(s)
                compute(s)
                fire_out(j, s)
            return 0
        lax.fori_loop(0, (RPT - 2) // 2, pair, 0)

        drain_out()
        drain_in(1)
        compute(1)
        fire_out(RPT - 1, 1)
        drain_out()
        plsc.subcore_barrier()

    def play_compute(s):
        def row(e, _):
            t = A[s][e, :] * B[s][e, :]
            B[s][e, :] = t
            A[s][e, :] = t * (1.0 + 0.1 * W1[s][e, :])
            return 0
        lax.fori_loop(0, 128, row, 0, unroll=2)

    def soc_compute(s):
        def row(e, _):
            m = A[s][e, :] * B[s][e, :]
            W1[s][e, :] = 0.2 * (W1[s][e, :] + W2[s][e, :]) * m
            W3[s][e, :] = 0.2 * (W3[s][e, :] + W4[s][e, :]) * m
            return 0
        lax.fori_loop(0, 128, row, 0, unroll=2)

    pltpu.sync_copy(play1.at[0, sl], iu)
    pltpu.sync_copy(play1.at[1, sl], ig)
    run_pass(invU1, invG1, (nw1,), (wA1, wB1), play_compute)
    pltpu.sync_copy(play2.at[0, sl], iu)
    pltpu.sync_copy(play2.at[1, sl], ig)
    run_pass(invU2, invG2, (nw2,), (wA2, wB2), play_compute)
    pltpu.sync_copy(soc.at[0, sl], iu)
    pltpu.sync_copy(soc.at[1, sl], ig)
    run_pass(invSO, invSI, (di1, ci1, di2, ci2), (wS1, wS2), soc_compute)


@functools.cache
def _weights_kernel_build():
    return pl.kernel(
        _weights_body,
        out_type=tuple(jax.ShapeDtypeStruct((EP, 16), jnp.float32)
                       for _ in range(6)),
        mesh=_mesh(),
        compiler_params=pltpu.CompilerParams(use_tc_tiling_on_sc=False),
        scratch_types=(
            [pltpu.VMEM((RPT, 128), jnp.int32)] * 2
            + [pltpu.VMEM((128, 16), jnp.float32)] * 12
            + [pltpu.VMEM_SHARED((NP, 16), jnp.float32)] * 2
            + [pltpu.SemaphoreType.DMA] * 3
        ),
    )


DH = 64
CK = 64
CPT = 10240 // CK
_RING = 4


def _spmm_phase(tabS, earr, scol, dcol, warr, wid, sidx, didx, rows, wv,
                acc, gsem, wsem, ssem):
    esl = pl.ds(wid * CPT, CPT)
    pltpu.sync_copy(earr.at[scol, esl], sidx)
    pltpu.sync_copy(earr.at[dcol, esl], didx)

    def fire_g(j, s):
        pltpu.async_copy(tabS.at[sidx.at[j]], rows[s], gsem)

    def fire_w(j, sw):
        pltpu.async_copy(warr.at[pl.ds((wid * CPT + j) * CK, CK)],
                         wv[sw], wsem)

    def drain_g(s):
        pltpu.make_async_copy(tabS.at[pl.ds(0, CK)], rows[s], gsem).wait()

    def drain_w(s):
        pltpu.make_async_copy(warr.at[pl.ds(0, CK)], wv[s], wsem).wait()

    def mul(s, sw):
        def mb(e, _):
            w = wv[sw][e, :]
            for q in range(DH // 16):
                sl16 = pl.ds(q * 16, 16)
                rows[s][e, sl16] = rows[s][e, sl16] * w
            return 0
        lax.fori_loop(0, CK, mb, 0, unroll=4)

    def fire_sc(j, s):
        pltpu.async_copy(rows[s], acc.at[didx.at[j]], ssem, add=True)

    def drain_sc():
        pltpu.make_async_copy(rows[0], acc.at[pl.ds(0, CK)], ssem).wait()

    def step(j, s, sw):
        drain_sc()
        drain_g(s)
        drain_w(sw)
        fire_g(j + 2, (s + 2) % 4)
        fire_w(j + 1, 1 - sw)
        mul(s, sw)
        fire_sc(j, s)

    fire_g(0, 0)
    fire_g(1, 1)
    fire_w(0, 0)
    drain_g(0)
    drain_w(0)
    fire_g(2, 2)
    fire_w(1, 1)
    mul(0, 0)
    fire_sc(0, 0)
    drain_g(1)
    drain_w(1)
    fire_g(3, 3)
    fire_w(2, 0)
    mul(1, 1)
    fire_sc(1, 1)

    def quad(t, _):
        j0 = 2 + t * 4
        for s4 in range(4):
            step(j0 + s4, (2 + s4) % 4, s4 % 2)
        return 0
    lax.fori_loop(0, (CPT - 4) // 4, quad, 0)

    drain_sc()
    drain_g(158 % 4)
    drain_w(0)
    fire_w(159, 1)
    mul(158 % 4, 0)
    fire_sc(158, 158 % 4)
    drain_sc()
    drain_g(159 % 4)
    drain_w(1)
    mul(159 % 4, 1)
    fire_sc(159, 159 % 4)
    drain_sc()
    drain_sc()


def _stage_tab(tab, tabS, sid):
    sl = pl.ds(sid * ART, ART)
    pltpu.sync_copy(tab.at[sl], tabS.at[sl])


def _spmm_half(phases, out, wid, sid, sidx, didx, rows, wv, acc, tabS,
               gsem, wsem, ssem):
    zero16 = jnp.zeros((16,), jnp.float32)
    r0 = rows[0]

    def zr(e, _):
        for q in range(DH // 16):
            r0[e, pl.ds(q * 16, 16)] = zero16
        return 0
    lax.fori_loop(0, CK, zr, 0)

    def za(k, _):
        pltpu.sync_copy(r0, acc.at[pl.ds(sid * ART + k * CK, CK)])
        return 0
    lax.fori_loop(0, ART // CK, za, 0)

    for (tab, earr, scol, dcol, warr) in phases:
        _stage_tab(tab, tabS, sid)
        plsc.subcore_barrier()
        _spmm_phase(tabS, earr, scol, dcol, warr, wid, sidx, didx, rows, wv,
                    acc, gsem, wsem, ssem)
        plsc.subcore_barrier()

    cid = lax.axis_index("c")
    pltpu.sync_copy(acc.at[pl.ds(sid * ART, ART)],
                    out.at[cid, pl.ds(sid * ART, ART)])
    plsc.subcore_barrier()


def _spmm_body(phasesL, phasesR, outL, outR, sidx, didx,
               r0, r1, r2, r3, w0, w1, acc, tabS, gsem, wsem, ssem):
    cid = lax.axis_index("c")
    sid = lax.axis_index("s")
    wid = cid * NS + sid
    rows = (r0, r1, r2, r3)
    wv = (w0, w1)
    _spmm_half(phasesL, outL, wid, sid, sidx, didx, rows, wv, acc, tabS,
               gsem, wsem, ssem)
    _spmm_half(phasesR, outR, wid, sid, sidx, didx, rows, wv, acc, tabS,
               gsem, wsem, ssem)


def _spmm2_body(hgL, hgR, huL, huR, e1, e2, w1, w2, outL, outR, *sc):
    _spmm_body([(hgL, e1, 1, 0, w1), (huL, e2, 0, 1, w2)],
               [(hgR, e1, 1, 0, w1), (huR, e2, 0, 1, w2)],
               outL, outR, *sc)


def _spmm1_body(huL, huR, e1, w1, outL, outR, *sc):
    _spmm_body([(huL, e1, 0, 1, w1)], [(huR, e1, 0, 1, w1)],
               outL, outR, *sc)


@functools.cache
def _spmm_build(nphases):
    body = _spmm2_body if nphases == 2 else _spmm1_body
    return pl.kernel(
        body,
        out_type=(jax.ShapeDtypeStruct((NC, NP, DH), jnp.float32),
                  jax.ShapeDtypeStruct((NC, NP, DH), jnp.float32)),
        mesh=_mesh(),
        compiler_params=pltpu.CompilerParams(use_tc_tiling_on_sc=False),
        scratch_types=(
            [pltpu.VMEM((CPT, CK), jnp.int32)] * 2
            + [pltpu.VMEM((CK, DH), jnp.float32)] * 4
            + [pltpu.VMEM((CK, 16), jnp.float32)] * 2
            + [pltpu.VMEM_SHARED((NP, DH), jnp.float32)]
            + [pltpu.VMEM_SHARED((NP, DH), jnp.float32)]
            + [pltpu.SemaphoreType.DMA] * 3
        ),
    )


def _invsq_body(c_ref, o_ref):
    o_ref[...] = jax.lax.rsqrt(jnp.maximum(c_ref[...], 1.0))


_invsq_tc = pl.pallas_call(
    _invsq_body,
    out_shape=jax.ShapeDtypeStruct((6 * NP * 16 // 128, 128), jnp.float32),
)


_BM = 1024


def _merge_body(ul_ref, ur_ref, gl_ref, gr_ref, hul, hur, hgl, hgr):
    hul[...] = ul_ref[0] + ul_ref[1]
    hur[...] = ur_ref[0] + ur_ref[1]
    hgl[...] = gl_ref[0] + gl_ref[1]
    hgr[...] = gr_ref[0] + gr_ref[1]


_merge = pl.pallas_call(
    _merge_body,
    grid=(NP // _BM,),
    in_specs=[pl.BlockSpec((NC, _BM, DH), lambda i: (0, i, 0))] * 4,
    out_specs=[pl.BlockSpec((_BM, DH), lambda i: (i, 0))] * 4,
    out_shape=[jax.ShapeDtypeStruct((NP, DH), jnp.float32)] * 4,
)


def _pad_edges(e):
    e = e.astype(jnp.int32)
    pad = jnp.full((2, EP - E), JUNK, jnp.int32)
    flat = jnp.concatenate([e, pad], axis=1)
    return flat.reshape(2, CR, 128), flat.reshape(2, EP // CK, CK)


def _rep_w(w):
    w = jnp.concatenate([w.astype(jnp.float32), jnp.zeros((EP - E,), jnp.float32)])
    return jnp.broadcast_to(w[:, None], (EP, 16))


def _pad_tab(t):
    return jnp.concatenate(
        [t.astype(jnp.float32), jnp.zeros((NP - N, D), jnp.float32)], axis=0)


def kernel(user_embedding, item_embedding, weight_noise_edge, weight_friend_of_DI,
           weight_friend_of_CI, dn_weight_noise_edge, dn_weight_friend_of_DI,
           dn_weight_friend_of_CI, edge_index_play, edge_index_social,
           edge_index_play_dn):
    p1, p1s = _pad_edges(edge_index_play)
    p2, p2s = _pad_edges(edge_index_play_dn)
    so, sos = _pad_edges(edge_index_social)
    nw1 = _rep_w(weight_noise_edge)
    di1 = _rep_w(weight_friend_of_DI)
    ci1 = _rep_w(weight_friend_of_CI)
    nw2 = _rep_w(dn_weight_noise_edge)
    di2 = _rep_w(dn_weight_friend_of_DI)
    ci2 = _rep_w(dn_weight_friend_of_CI)

    counts = _deg_kernel_build()(p1, p2, so)
    invsq = _invsq_tc(counts.reshape(6 * NP * 16 // 128, 128)).reshape(6, NP, 16)
    wA1, wB1, wA2, wB2, wS1, wS2 = _weights_kernel_build()(
        invsq[0], invsq[1], invsq[2], invsq[3], invsq[4], invsq[5],
        p1, p2, so, nw1, di1, ci1, nw2, di2, ci2)
    _spmm2 = _spmm_build(2)
    _spmm1 = _spmm_build(1)

    hu0 = _pad_tab(user_embedding)
    hg0 = _pad_tab(item_embedding)
    outs = []
    for (ps, wA, wB, wS) in ((p1s, wA1, wB1, wS1), (p2s, wA2, wB2, wS2)):
        huL, huR = hu0[:, :DH], hu0[:, DH:]
        hgL, hgR = hg0[:, :DH], hg0[:, DH:]
        for _ in range(3):
            upL, upR = _spmm2(hgL, hgR, huL, huR, ps, sos, wA, wS)
            gpL, gpR = _spmm1(huL, huR, ps, wB)
            huL, huR, hgL, hgR = _merge(upL, upR, gpL, gpR)
        outs += [jnp.concatenate([huL[:N], huR[:N]], axis=1),
                 jnp.concatenate([hgL[:N], hgR[:N]], axis=1)]
    return jnp.stack(outs)
